# baseline (speedup 1.0000x reference)
import functools

import jax
import jax.numpy as jnp
from jax import lax
from jax.experimental import pallas as pl
from jax.experimental.pallas import tpu as pltpu
from jax.experimental.pallas import tpu_sc as plsc


def _make_sc_gather(n_idx, n_chunks, chunk, c):
    info = plsc.get_sparse_core_info()
    nc, ns = info.num_cores, info.num_subcores
    nw = nc * ns
    per_w = n_chunks // nw
    mesh = plsc.VectorSubcoreMesh(core_axis_name="c", subcore_axis_name="s")

    @functools.partial(
        pl.kernel,
        mesh=mesh,
        out_type=(
            jax.ShapeDtypeStruct((n_idx, c), jnp.float32),
            jax.ShapeDtypeStruct((n_idx, c), jnp.float32),
        ),
        scratch_types=[
            pltpu.VMEM((chunk,), jnp.int32),
            pltpu.VMEM((chunk,), jnp.int32),
            pltpu.VMEM((chunk, c), jnp.float32),
            pltpu.VMEM((chunk, c), jnp.float32),
            pltpu.SemaphoreType.DMA,
            pltpu.SemaphoreType.DMA,
        ],
    )
    def sc_gather(ph_hbm, pw_hbm, th_hbm, tw_hbm, eh_hbm, ew_hbm,
                  idxh_v, idxw_v, rowsh_v, rowsw_v, semh, semw):
        wid = lax.axis_index("s") * nc + lax.axis_index("c")
        for j in range(per_w):
            row = wid * per_w + j
            base = row * chunk
            pltpu.sync_copy(ph_hbm.at[row], idxh_v)
            pltpu.sync_copy(pw_hbm.at[row], idxw_v)
            ch = pltpu.async_copy(th_hbm.at[idxh_v], rowsh_v, semh)
            cw = pltpu.async_copy(tw_hbm.at[idxw_v], rowsw_v, semw)
            ch.wait()
            cw.wait()
            pltpu.sync_copy(rowsh_v, eh_hbm.at[pl.ds(base, chunk)])
            pltpu.sync_copy(rowsw_v, ew_hbm.at[pl.ds(base, chunk)])

    return sc_gather


def _stripe_kernel(eh_ref, ew_ref, x_ref, o_ref):
    c = x_ref.shape[1]
    w0 = eh_ref.shape[1]
    w = x_ref.shape[3]
    s = eh_ref[0] + ew_ref[0]
    ups = (
        lax.broadcasted_iota(jnp.int32, (w0, w), 0)
        == lax.broadcasted_iota(jnp.int32, (w0, w), 1) // (w // w0)
    ).astype(jnp.float32)
    a = lax.dot_general(
        s, ups, (((0,), (0,)), ((), ())), preferred_element_type=jnp.float32
    )
    o_ref[0] = x_ref[0] + a[:c, None, :]


def kernel(x, pos_h, pos_w, table_h, table_w):
    B, C, H, W = x.shape
    H0, W0 = pos_h.shape[1], pos_h.shape[2]
    hb = H // H0
    n_idx = B * H0 * W0
    chunk = 128
    n_chunks = n_idx // chunk

    ph = pos_h.reshape(n_chunks, chunk)
    pw = pos_w.reshape(n_chunks, chunk)
    c_pad = 128
    th = jnp.pad(table_h, ((0, 0), (0, c_pad - C)))
    tw = jnp.pad(table_w, ((0, 0), (0, c_pad - C)))
    eh, ew = _make_sc_gather(n_idx, n_chunks, chunk, c_pad)(ph, pw, th, tw)
    eh = eh.reshape(B * H0, W0, c_pad)
    ew = ew.reshape(B * H0, W0, c_pad)

    grid = (B * H0,)
    return pl.pallas_call(
        _stripe_kernel,
        grid=grid,
        in_specs=[
            pl.BlockSpec((1, W0, 128), lambda i: (i, 0, 0)),
            pl.BlockSpec((1, W0, 128), lambda i: (i, 0, 0)),
            pl.BlockSpec((1, C, hb, W), lambda i: (i // H0, 0, i % H0, 0)),
        ],
        out_specs=pl.BlockSpec((1, C, hb, W), lambda i: (i // H0, 0, i % H0, 0)),
        out_shape=jax.ShapeDtypeStruct(x.shape, x.dtype),
    )(eh, ew, x)
